# baseline (speedup 1.0000x reference)
import functools

import jax
import jax.numpy as jnp
import numpy as np
from jax import lax
from jax.experimental import pallas as pl
from jax.experimental.pallas import tpu as pltpu
from jax.experimental.pallas import tpu_sc as plsc

_DIAGONAL_OFFSET = 2

_B = 4
_UT_LEN = 130305
_D = 64
_TOTAL = _B * _UT_LEN

_SEG = 128
_GROUP_SEGS = 8
_CHUNK = _SEG * _GROUP_SEGS
_NGROUP_FULL = _TOTAL // _CHUNK
_TAIL_ROWS = _TOTAL - _NGROUP_FULL * _CHUNK
_NGROUP_PAD = _NGROUP_FULL + 1
_NW = 32
_TAIL_OWNER = _NGROUP_FULL % _NW


def _reverse_perm(ut_len, diagonal_offset):
    seq_len = int(np.sqrt(2 * ut_len + 0.25) - 0.5) + diagonal_offset
    ut_indexes = np.triu_indices(seq_len, diagonal_offset)
    assert len(ut_indexes[0]) == ut_len
    mat_ut_indexes = np.zeros(shape=(seq_len, seq_len), dtype="int")
    mat_ut_indexes[ut_indexes] = np.arange(ut_len)
    mask_ut = np.zeros(shape=(seq_len, seq_len), dtype="bool")
    mask_ut[ut_indexes] = True
    mat_indexes = mat_ut_indexes + np.multiply(~mask_ut, mat_ut_indexes.T)
    mat_rc_indexes = mat_indexes[::-1, ::-1]
    return mat_rc_indexes[ut_indexes]


@functools.lru_cache(maxsize=None)
def _gather_indices():
    perm = _reverse_perm(_UT_LEN, _DIAGONAL_OFFSET).astype(np.int64)
    base = (np.arange(_B, dtype=np.int64)[:, None] * _UT_LEN + perm[None, :]).reshape(-1)
    pad = _NGROUP_PAD * _CHUNK - _TOTAL
    base = np.concatenate([base, np.zeros((pad,), dtype=np.int64)])
    return base.reshape(_NGROUP_PAD * _GROUP_SEGS, _SEG).astype(np.int32)


def _sc_body(x_hbm, idx_hbm, rev_hbm, out_hbm, idx_v, rows_v, rev_v, sem):
    nc = lax.axis_size("c")
    wid = lax.axis_index("s") * nc + lax.axis_index("c")

    pltpu.sync_copy(rev_hbm, rev_v)
    rev = rev_v[...][0]

    nt = (_NGROUP_FULL - wid + (_NW - 1)) // _NW

    def gather_group(j):
        pltpu.sync_copy(idx_hbm.at[pl.ds(j * _GROUP_SEGS, _GROUP_SEGS)], idx_v)
        handles = [
            pltpu.async_copy(
                x_hbm.at[idx_v.at[i]],
                rows_v.at[pl.ds(i * _SEG, _SEG)],
                sem,
            )
            for i in range(_GROUP_SEGS)
        ]
        for h in handles:
            h.wait()
        pltpu.sync_copy(rows_v, out_hbm.at[pl.ds(j * _CHUNK, _CHUNK)])

    def copy_group(j):
        pltpu.sync_copy(x_hbm.at[pl.ds(j * _CHUNK, _CHUNK)], rows_v)
        pltpu.sync_copy(rows_v, out_hbm.at[pl.ds(j * _CHUNK, _CHUNK)])

    @pl.when(rev != 0)
    def _():
        def body(t, carry):
            gather_group(wid + t * _NW)
            return carry

        lax.fori_loop(0, nt, body, 0)

        @pl.when(wid == _TAIL_OWNER)
        def _():
            j = _NGROUP_FULL
            pltpu.sync_copy(idx_hbm.at[pl.ds(j * _GROUP_SEGS, 1)],
                            idx_v.at[pl.ds(0, 1)])
            pltpu.async_copy(
                x_hbm.at[idx_v.at[0]], rows_v.at[pl.ds(0, _SEG)], sem
            ).wait()
            pltpu.sync_copy(rows_v.at[pl.ds(0, _TAIL_ROWS)],
                            out_hbm.at[pl.ds(j * _CHUNK, _TAIL_ROWS)])

    @pl.when(rev == 0)
    def _():
        def body(t, carry):
            copy_group(wid + t * _NW)
            return carry

        lax.fori_loop(0, nt, body, 0)

        @pl.when(wid == _TAIL_OWNER)
        def _():
            j = _NGROUP_FULL
            pltpu.sync_copy(x_hbm.at[pl.ds(j * _CHUNK, _TAIL_ROWS)],
                            rows_v.at[pl.ds(0, _TAIL_ROWS)])
            pltpu.sync_copy(rows_v.at[pl.ds(0, _TAIL_ROWS)],
                            out_hbm.at[pl.ds(j * _CHUNK, _TAIL_ROWS)])


@jax.jit
def _sc_gather(x_flat, idx, rev_vec):
    call = pl.kernel(
        _sc_body,
        out_type=jax.ShapeDtypeStruct((_TOTAL, _D), jnp.float32),
        mesh=plsc.VectorSubcoreMesh(core_axis_name="c", subcore_axis_name="s"),
        scratch_types=[
            pltpu.VMEM((_GROUP_SEGS, _SEG), jnp.int32),
            pltpu.VMEM((_CHUNK, _D), jnp.float32),
            pltpu.VMEM((16,), jnp.int32),
            pltpu.SemaphoreType.DMA,
        ],
        compiler_params=pltpu.CompilerParams(use_tc_tiling_on_sc=False),
    )
    return call(x_flat, idx, rev_vec)


def kernel(x_ut, reverse):
    assert x_ut.shape == (_B, _UT_LEN, _D), x_ut.shape
    x_flat = x_ut.reshape(_TOTAL, _D)
    idx = jnp.asarray(_gather_indices())
    rev_vec = jnp.broadcast_to(jnp.asarray(reverse, jnp.int32), (16,))
    out_flat = _sc_gather(x_flat, idx, rev_vec)
    return out_flat.reshape(_B, _UT_LEN, _D)


# speedup vs baseline: 7.8336x; 7.8336x over previous
import functools

import jax
import jax.numpy as jnp
import numpy as np
from jax import lax
from jax.experimental import pallas as pl
from jax.experimental.pallas import tpu as pltpu
from jax.experimental.pallas import tpu_sc as plsc

_DIAGONAL_OFFSET = 2

_B = 4
_UT_LEN = 130305
_D = 64
_ROW = _B * _D

_SEG = 128
_NFULL = _UT_LEN // _SEG
_LAST_BASE = _UT_LEN - _SEG
_NW = 32
_GPW = 32
_NIDX = 1024


def _reverse_perm(ut_len, diagonal_offset):
    seq_len = int(np.sqrt(2 * ut_len + 0.25) - 0.5) + diagonal_offset
    ut_indexes = np.triu_indices(seq_len, diagonal_offset)
    assert len(ut_indexes[0]) == ut_len
    mat_ut_indexes = np.zeros(shape=(seq_len, seq_len), dtype="int")
    mat_ut_indexes[ut_indexes] = np.arange(ut_len)
    mask_ut = np.zeros(shape=(seq_len, seq_len), dtype="bool")
    mask_ut[ut_indexes] = True
    mat_indexes = mat_ut_indexes + np.multiply(~mask_ut, mat_ut_indexes.T)
    mat_rc_indexes = mat_indexes[::-1, ::-1]
    return mat_rc_indexes[ut_indexes]


@functools.lru_cache(maxsize=None)
def _index_planes():
    perm = _reverse_perm(_UT_LEN, _DIAGONAL_OFFSET).astype(np.int32)
    iota = np.arange(_UT_LEN, dtype=np.int32)
    planes = np.zeros((2, _NIDX, _SEG), dtype=np.int32)
    for p, src in ((0, iota), (1, perm)):
        planes[p, :_NFULL] = src[: _NFULL * _SEG].reshape(_NFULL, _SEG)
        planes[p, _NFULL] = src[_LAST_BASE:]
    return planes


def _sc_body(x_hbm, idx_hbm, rev_hbm, out_hbm, idx_v, dst_v, rows_v, rev_v, sem):
    nc = lax.axis_size("c")
    wid = lax.axis_index("s") * nc + lax.axis_index("c")

    pltpu.sync_copy(rev_hbm, rev_v)
    rev = rev_v[...][0]
    plane = jnp.where(rev != 0, 1, 0)

    base_g = wid * _GPW
    nt = jnp.minimum(_GPW, _NFULL - base_g)

    pltpu.sync_copy(idx_hbm.at[plane, pl.ds(base_g, _GPW)], idx_v)

    def body(t, carry):
        pltpu.async_copy(x_hbm.at[idx_v.at[t]], rows_v, sem).wait()
        pltpu.sync_copy(rows_v, out_hbm.at[pl.ds((base_g + t) * _SEG, _SEG)])
        return carry

    lax.fori_loop(0, nt, body, 0)

    @pl.when(wid == _NW - 1)
    def _():
        t_last = _NFULL - (_NW - 1) * _GPW
        dst_base = (_NFULL // 8) * 8
        pltpu.sync_copy(idx_hbm.at[0, pl.ds(dst_base, 8)], dst_v)
        pltpu.async_copy(x_hbm.at[idx_v.at[t_last]], rows_v, sem).wait()
        pltpu.async_copy(rows_v, out_hbm.at[dst_v.at[_NFULL - dst_base]], sem).wait()


@jax.jit
def _sc_gather(xt, idx, rev_vec):
    call = pl.kernel(
        _sc_body,
        out_type=jax.ShapeDtypeStruct((_UT_LEN, _ROW), jnp.float32),
        mesh=plsc.VectorSubcoreMesh(core_axis_name="c", subcore_axis_name="s"),
        scratch_types=[
            pltpu.VMEM((_GPW, _SEG), jnp.int32),
            pltpu.VMEM((8, _SEG), jnp.int32),
            pltpu.VMEM((_SEG, _ROW), jnp.float32),
            pltpu.VMEM((16,), jnp.int32),
            pltpu.SemaphoreType.DMA,
        ],
    )
    return call(xt, idx, rev_vec)


def kernel(x_ut, reverse):
    assert x_ut.shape == (_B, _UT_LEN, _D), x_ut.shape
    xt = jnp.transpose(x_ut, (1, 0, 2)).reshape(_UT_LEN, _ROW)
    idx = jnp.asarray(_index_planes())
    rev_vec = jnp.broadcast_to(jnp.asarray(reverse, jnp.int32), (16,))
    out = _sc_gather(xt, idx, rev_vec)
    return jnp.transpose(out.reshape(_UT_LEN, _B, _D), (1, 0, 2))
